# baseline (speedup 1.0000x reference)
import functools

import jax
import jax.numpy as jnp
from jax import lax
from jax.experimental import pallas as pl
from jax.experimental.pallas import tpu as pltpu
from jax.experimental.pallas import tpu_sc as plsc

N_NODES = 10000
N_EDGES = 320000
NW = 32
CHUNK = 128
NCF = 160
QUARTER = NCF // 4
NCHUNK = 80
EPW = NCHUNK * CHUNK
E_PAD = NW * EPW
NPAD = 10240
DUMMY = N_NODES
ROWS_PW = NPAD // 16
PACK_SHIFT = 14
PACK_MASK = (1 << PACK_SHIFT) - 1

_MESH = plsc.VectorSubcoreMesh(core_axis_name="c", subcore_axis_name="s")
_SC_PARAMS = pltpu.CompilerParams(needs_layout_passes=False)


@functools.partial(
    pl.kernel,
    mesh=_MESH,
    out_type=jax.ShapeDtypeStruct((NW, NPAD), jnp.float32),
    scratch_types=[
        pltpu.VMEM((EPW,), jnp.int32),
        pltpu.VMEM((NPAD,), jnp.float32),
    ],
    compiler_params=_SC_PARAMS,
)
def _deg_kernel(pk_hbm, deg_out, pk_v, deg_v):
    c = lax.axis_index("c")
    s = lax.axis_index("s")
    wid = s * 2 + c
    pltpu.sync_copy(pk_hbm.at[wid], pk_v)

    zeros16 = jnp.zeros((16,), jnp.float32)
    ones16 = jnp.ones((16,), jnp.float32)
    dummy16 = jnp.full((16,), DUMMY, jnp.int32)
    mask16 = jnp.full((16,), PACK_MASK, jnp.int32)

    def zero_body(i, carry):
        deg_v[pl.ds(i * 16, 16)] = zeros16
        return carry

    lax.fori_loop(0, NPAD // 16, zero_body, 0)

    def add_body(g, carry):
        pk16 = pk_v[pl.ds(g * 16, 16)]
        r16 = lax.shift_right_logical(pk16, PACK_SHIFT)
        c16 = lax.bitwise_and(pk16, mask16)
        idx = jnp.where(r16 == c16, dummy16, r16)
        plsc.addupdate_scatter(deg_v, [idx], ones16)
        return carry

    lax.fori_loop(0, EPW // 16, add_body, 0)
    pltpu.sync_copy(deg_v, deg_out.at[wid])


def _make_scatter_kernel(d):
    @functools.partial(
        pl.kernel,
        mesh=_MESH,
        out_type=jax.ShapeDtypeStruct((1, NPAD, d), jnp.float32),
        scratch_types=[
            pltpu.VMEM((QUARTER, CHUNK), jnp.int32),
            pltpu.VMEM((QUARTER, CHUNK), jnp.int32),
            pltpu.VMEM((CHUNK, d), jnp.float32),
            pltpu.VMEM((CHUNK, d), jnp.float32),
            pltpu.VMEM_SHARED((NPAD, d), jnp.float32),
            pltpu.SemaphoreType.DMA,
            pltpu.SemaphoreType.DMA,
        ],
        compiler_params=_SC_PARAMS,
    )
    def _scatter_kernel(u_hbm, pk_hbm, acc_out,
                        row_v, col_v, g0, g1, acc_s, sem0, sem1):
        c = lax.axis_index("c")
        s = lax.axis_index("s")

        zeros16 = jnp.zeros((16,), jnp.float32)
        dummy16 = jnp.full((16,), DUMMY, jnp.int32)
        mask16 = jnp.full((16,), PACK_MASK, jnp.int32)

        @pl.when(c == 0)
        def _():
            def gz_body(r, carry):
                for k in range(d // 16):
                    g0[r, pl.ds(k * 16, 16)] = zeros16
                return carry

            lax.fori_loop(0, CHUNK, gz_body, 0)
            for z in range(ROWS_PW // CHUNK):
                pltpu.sync_copy(g0,
                                acc_s.at[pl.ds(s * ROWS_PW + z * CHUNK, CHUNK)])
            plsc.subcore_barrier()

            for h in range(4):
                pltpu.sync_copy(pk_hbm.at[s].at[pl.ds(h * QUARTER, QUARTER)],
                                row_v)

                def unpack_body(j, carry):
                    for k in range(CHUNK // 16):
                        pk16 = row_v[j, pl.ds(k * 16, 16)]
                        r16 = lax.shift_right_logical(pk16, PACK_SHIFT)
                        c16 = lax.bitwise_and(pk16, mask16)
                        row_v[j, pl.ds(k * 16, 16)] = r16
                        col_v[j, pl.ds(k * 16, 16)] = jnp.where(
                            r16 == c16, dummy16, c16)
                    return carry

                lax.fori_loop(0, QUARTER, unpack_body, 0)

                def chunk_body(jj, carry):
                    j0 = 2 * jj
                    j1 = 2 * jj + 1
                    cp0 = pltpu.async_copy(u_hbm.at[row_v.at[j0]], g0, sem0)
                    cp1 = pltpu.async_copy(u_hbm.at[row_v.at[j1]], g1, sem1)
                    cp0.wait()
                    pltpu.sync_copy(g0, acc_s.at[col_v.at[j0]], add=True)
                    cp1.wait()
                    pltpu.sync_copy(g1, acc_s.at[col_v.at[j1]], add=True)
                    return carry

                lax.fori_loop(0, QUARTER // 2, chunk_body, 0)

            plsc.subcore_barrier()
            pltpu.sync_copy(acc_s.at[pl.ds(s * ROWS_PW, ROWS_PW)],
                            acc_out.at[0].at[pl.ds(s * ROWS_PW, ROWS_PW)])

    return _scatter_kernel


_scatter_128 = _make_scatter_kernel(128)


_BLK = 1024
_GRID = NPAD // _BLK


def _dinv_of(dp_block):
    deg = jnp.sum(dp_block, axis=0)
    return jnp.where(deg > 0, lax.rsqrt(deg), 0.0)


def _b_body(x_ref, dp_ref, w10_ref, w11_ref, xw10_ref, u1_ref):
    dinv = _dinv_of(dp_ref[...])
    xb = x_ref[...]
    xw10_ref[...] = jnp.dot(xb, w10_ref[...], preferred_element_type=jnp.float32)
    u1_ref[...] = dinv[:, None] * jnp.dot(xb, w11_ref[...],
                                          preferred_element_type=jnp.float32)


def _d1_body(xw10_ref, acc_ref, dp_ref, b1_ref, w20_ref, w21_ref,
             y1w20_ref, u2_ref):
    dinv = _dinv_of(dp_ref[...])
    accsum = acc_ref[0]
    y1 = jax.nn.relu(xw10_ref[...] - dinv[:, None] * accsum + b1_ref[...])
    y1w20_ref[...] = jnp.dot(y1, w20_ref[...], preferred_element_type=jnp.float32)
    u2_ref[...] = dinv[:, None] * jnp.dot(y1, w21_ref[...],
                                          preferred_element_type=jnp.float32)


def _d2_body(y1w20_ref, acc_ref, dp_ref, b2_ref, wl_ref, bl_ref, out_ref):
    dinv = _dinv_of(dp_ref[...])
    accsum = acc_ref[0]
    y2 = jax.nn.relu(y1w20_ref[...] - dinv[:, None] * accsum + b2_ref[...])
    out_ref[...] = jnp.dot(y2, wl_ref[...],
                           preferred_element_type=jnp.float32) + bl_ref[...]


def _full(shape):
    return pl.BlockSpec(shape, lambda i: (0,) * len(shape))


def _rows(d):
    return pl.BlockSpec((_BLK, d), lambda i: (i, 0))


_DP_SPEC = pl.BlockSpec((NW, _BLK), lambda i: (0, i))


def _acc_spec(d):
    return pl.BlockSpec((1, _BLK, d), lambda i: (0, i, 0))


def kernel(x, edge_index, W10, W11, b1, W20, W21, b2, Wl, bl):
    f32 = jnp.float32
    row = edge_index[0].astype(jnp.int32)
    col = edge_index[1].astype(jnp.int32)
    pad = E_PAD - N_EDGES
    packed = jnp.concatenate(
        [row * (PACK_MASK + 1) + col, jnp.zeros((pad,), jnp.int32)])
    pk2 = packed.reshape(NW, EPW)
    pk3 = packed.reshape(16, NCF, CHUNK)

    W10p = jnp.pad(W10, ((0, 0), (0, 28)))
    W11p = jnp.pad(W11, ((0, 0), (0, 28)))
    b1p = jnp.pad(b1, (0, 28)).reshape(1, 128)
    W20p = jnp.pad(W20, ((0, 28), (0, 0)))
    W21p = jnp.pad(W21, ((0, 28), (0, 0)))
    b2p = b2.reshape(1, 128)
    blp = bl.reshape(1, 512)

    deg_parts = _deg_kernel(pk2)

    x_p = jnp.pad(x, ((0, NPAD - N_NODES), (0, 0)))
    xw10, u1 = pl.pallas_call(
        _b_body,
        grid=(_GRID,),
        in_specs=[_rows(128), _DP_SPEC, _full((128, 128)), _full((128, 128))],
        out_specs=[_rows(128), _rows(128)],
        out_shape=[jax.ShapeDtypeStruct((NPAD, 128), f32)] * 2,
    )(x_p, deg_parts, W10p, W11p)

    acc1 = _scatter_128(u1, pk3)

    y1w20, u2 = pl.pallas_call(
        _d1_body,
        grid=(_GRID,),
        in_specs=[_rows(128), _acc_spec(128), _DP_SPEC, _full((1, 128)),
                  _full((128, 128)), _full((128, 128))],
        out_specs=[_rows(128), _rows(128)],
        out_shape=[jax.ShapeDtypeStruct((NPAD, 128), f32)] * 2,
    )(xw10, acc1, deg_parts, b1p, W20p, W21p)

    acc2 = _scatter_128(u2, pk3)

    out = pl.pallas_call(
        _d2_body,
        grid=(_GRID,),
        in_specs=[_rows(128), _acc_spec(128), _DP_SPEC, _full((1, 128)),
                  _full((128, 512)), _full((1, 512))],
        out_specs=_rows(512),
        out_shape=jax.ShapeDtypeStruct((NPAD, 512), f32),
    )(y1w20, acc2, deg_parts, b2p, Wl, blp)

    return out[:N_NODES]


# speedup vs baseline: 1.4932x; 1.4932x over previous
import functools

import jax
import jax.numpy as jnp
from jax import lax
from jax.experimental import pallas as pl
from jax.experimental.pallas import tpu as pltpu
from jax.experimental.pallas import tpu_sc as plsc

N_NODES = 10000
N_EDGES = 320000
NW = 32
CHUNK = 128
NC0 = 112
NC1 = 48
NCHUNK = (NC0 + NC1) // 2
NCMAX = max(NC0, NC1)
HALF = NCMAX // 2
EPW = NCHUNK * CHUNK
E_PAD = NW * EPW
NPAD = 10240
DUMMY = N_NODES
ROWS_PW = NPAD // 16
PACK_SHIFT = 14
PACK_MASK = (1 << PACK_SHIFT) - 1

_MESH = plsc.VectorSubcoreMesh(core_axis_name="c", subcore_axis_name="s")
_SC_PARAMS = pltpu.CompilerParams(needs_layout_passes=False)


@functools.partial(
    pl.kernel,
    mesh=_MESH,
    out_type=jax.ShapeDtypeStruct((NW, NPAD), jnp.float32),
    scratch_types=[
        pltpu.VMEM((EPW,), jnp.int32),
        pltpu.VMEM((NPAD,), jnp.float32),
    ],
    compiler_params=_SC_PARAMS,
)
def _deg_kernel(pk_hbm, deg_out, pk_v, deg_v):
    c = lax.axis_index("c")
    s = lax.axis_index("s")
    wid = s * 2 + c
    pltpu.sync_copy(pk_hbm.at[wid], pk_v)

    zeros16 = jnp.zeros((16,), jnp.float32)
    ones16 = jnp.ones((16,), jnp.float32)
    dummy16 = jnp.full((16,), DUMMY, jnp.int32)
    mask16 = jnp.full((16,), PACK_MASK, jnp.int32)

    def zero_body(i, carry):
        deg_v[pl.ds(i * 16, 16)] = zeros16
        return carry

    lax.fori_loop(0, NPAD // 16, zero_body, 0)

    def add_body(g, carry):
        pk16 = pk_v[pl.ds(g * 16, 16)]
        r16 = lax.shift_right_logical(pk16, PACK_SHIFT)
        c16 = lax.bitwise_and(pk16, mask16)
        idx = jnp.where(r16 == c16, dummy16, r16)
        plsc.addupdate_scatter(deg_v, [idx], ones16)
        return carry

    lax.fori_loop(0, EPW // 16, add_body, 0)
    pltpu.sync_copy(deg_v, deg_out.at[wid])


def _make_scatter_kernel(d):
    @functools.partial(
        pl.kernel,
        mesh=_MESH,
        out_type=jax.ShapeDtypeStruct((2, NPAD, d), jnp.float32),
        scratch_types=[
            pltpu.VMEM((HALF, CHUNK), jnp.int32),
            pltpu.VMEM((HALF, CHUNK), jnp.int32),
            pltpu.VMEM((CHUNK, d), jnp.float32),
            pltpu.VMEM((CHUNK, d), jnp.float32),
            pltpu.VMEM_SHARED((NPAD, d), jnp.float32),
            pltpu.SemaphoreType.DMA,
            pltpu.SemaphoreType.DMA,
        ],
        compiler_params=_SC_PARAMS,
    )
    def _scatter_kernel(u_hbm, pk_hbm, acc_out,
                        row_v, col_v, g0, g1, acc_s, sem0, sem1):
        c = lax.axis_index("c")
        s = lax.axis_index("s")
        wid = s * 2 + c

        zeros16 = jnp.zeros((16,), jnp.float32)
        dummy16 = jnp.full((16,), DUMMY, jnp.int32)
        mask16 = jnp.full((16,), PACK_MASK, jnp.int32)

        def gz_body(r, carry):
            for k in range(d // 16):
                g0[r, pl.ds(k * 16, 16)] = zeros16
            return carry

        lax.fori_loop(0, CHUNK, gz_body, 0)
        for z in range(ROWS_PW // CHUNK):
            pltpu.sync_copy(g0, acc_s.at[pl.ds(s * ROWS_PW + z * CHUNK, CHUNK)])
        plsc.subcore_barrier()

        def stream(nc):
            half = nc // 2
            for h in range(2):
                pltpu.sync_copy(pk_hbm.at[wid].at[pl.ds(h * half, half)],
                                row_v.at[pl.ds(0, half)])

                def unpack_body(j, carry):
                    for k in range(CHUNK // 16):
                        pk16 = row_v[j, pl.ds(k * 16, 16)]
                        r16 = lax.shift_right_logical(pk16, PACK_SHIFT)
                        c16 = lax.bitwise_and(pk16, mask16)
                        row_v[j, pl.ds(k * 16, 16)] = r16
                        col_v[j, pl.ds(k * 16, 16)] = jnp.where(
                            r16 == c16, dummy16, c16)
                    return carry

                lax.fori_loop(0, half, unpack_body, 0)

                def chunk_body(jj, carry):
                    j0 = 2 * jj
                    j1 = 2 * jj + 1
                    cp0 = pltpu.async_copy(u_hbm.at[row_v.at[j0]], g0, sem0)
                    cp1 = pltpu.async_copy(u_hbm.at[row_v.at[j1]], g1, sem1)
                    cp0.wait()
                    pltpu.sync_copy(g0, acc_s.at[col_v.at[j0]], add=True)
                    cp1.wait()
                    pltpu.sync_copy(g1, acc_s.at[col_v.at[j1]], add=True)
                    return carry

                lax.fori_loop(0, half // 2, chunk_body, 0)

        @pl.when(c == 0)
        def _():
            stream(NC0)

        @pl.when(c == 1)
        def _():
            stream(NC1)

        plsc.subcore_barrier()
        pltpu.sync_copy(acc_s.at[pl.ds(s * ROWS_PW, ROWS_PW)],
                        acc_out.at[c].at[pl.ds(s * ROWS_PW, ROWS_PW)])

    return _scatter_kernel


_scatter_128 = _make_scatter_kernel(128)


_BLK = 1024
_GRID = NPAD // _BLK


def _dinv_of(dp_block):
    deg = jnp.sum(dp_block, axis=0)
    return jnp.where(deg > 0, lax.rsqrt(deg), 0.0)


def _b_body(x_ref, dp_ref, w10_ref, w11_ref, xw10_ref, u1_ref):
    dinv = _dinv_of(dp_ref[...])
    xb = x_ref[...]
    xw10_ref[...] = jnp.dot(xb, w10_ref[...], preferred_element_type=jnp.float32)
    u1_ref[...] = dinv[:, None] * jnp.dot(xb, w11_ref[...],
                                          preferred_element_type=jnp.float32)


def _d1_body(xw10_ref, acc_ref, dp_ref, b1_ref, w20_ref, w21_ref,
             y1w20_ref, u2_ref):
    dinv = _dinv_of(dp_ref[...])
    accsum = acc_ref[0] + acc_ref[1]
    y1 = jax.nn.relu(xw10_ref[...] - dinv[:, None] * accsum + b1_ref[...])
    y1w20_ref[...] = jnp.dot(y1, w20_ref[...], preferred_element_type=jnp.float32)
    u2_ref[...] = dinv[:, None] * jnp.dot(y1, w21_ref[...],
                                          preferred_element_type=jnp.float32)


def _d2_body(y1w20_ref, acc_ref, dp_ref, b2_ref, wl_ref, bl_ref, out_ref):
    dinv = _dinv_of(dp_ref[...])
    accsum = acc_ref[0] + acc_ref[1]
    y2 = jax.nn.relu(y1w20_ref[...] - dinv[:, None] * accsum + b2_ref[...])
    out_ref[...] = jnp.dot(y2, wl_ref[...],
                           preferred_element_type=jnp.float32) + bl_ref[...]


def _full(shape):
    return pl.BlockSpec(shape, lambda i: (0,) * len(shape))


def _rows(d):
    return pl.BlockSpec((_BLK, d), lambda i: (i, 0))


_DP_SPEC = pl.BlockSpec((NW, _BLK), lambda i: (0, i))


def _acc_spec(d):
    return pl.BlockSpec((2, _BLK, d), lambda i: (0, i, 0))


def kernel(x, edge_index, W10, W11, b1, W20, W21, b2, Wl, bl):
    f32 = jnp.float32
    row = edge_index[0].astype(jnp.int32)
    col = edge_index[1].astype(jnp.int32)
    pad = E_PAD - N_EDGES
    packed = jnp.concatenate(
        [row * (PACK_MASK + 1) + col, jnp.zeros((pad,), jnp.int32)])
    pk2 = packed.reshape(NW, EPW)
    pk_c0 = packed[:16 * NC0 * CHUNK].reshape(16, NC0, CHUNK)
    pk_c1 = packed[16 * NC0 * CHUNK:].reshape(16, NC1, CHUNK)
    pk_c0 = jnp.pad(pk_c0, ((0, 0), (0, NCMAX - NC0), (0, 0)))
    pk_c1 = jnp.pad(pk_c1, ((0, 0), (0, NCMAX - NC1), (0, 0)))
    pk3 = jnp.stack([pk_c0, pk_c1], axis=1).reshape(NW, NCMAX, CHUNK)

    W10p = jnp.pad(W10, ((0, 0), (0, 28)))
    W11p = jnp.pad(W11, ((0, 0), (0, 28)))
    b1p = jnp.pad(b1, (0, 28)).reshape(1, 128)
    W20p = jnp.pad(W20, ((0, 28), (0, 0)))
    W21p = jnp.pad(W21, ((0, 28), (0, 0)))
    b2p = b2.reshape(1, 128)
    blp = bl.reshape(1, 512)

    deg_parts = _deg_kernel(pk2)

    x_p = jnp.pad(x, ((0, NPAD - N_NODES), (0, 0)))
    xw10, u1 = pl.pallas_call(
        _b_body,
        grid=(_GRID,),
        in_specs=[_rows(128), _DP_SPEC, _full((128, 128)), _full((128, 128))],
        out_specs=[_rows(128), _rows(128)],
        out_shape=[jax.ShapeDtypeStruct((NPAD, 128), f32)] * 2,
    )(x_p, deg_parts, W10p, W11p)

    acc1 = _scatter_128(u1, pk3)

    y1w20, u2 = pl.pallas_call(
        _d1_body,
        grid=(_GRID,),
        in_specs=[_rows(128), _acc_spec(128), _DP_SPEC, _full((1, 128)),
                  _full((128, 128)), _full((128, 128))],
        out_specs=[_rows(128), _rows(128)],
        out_shape=[jax.ShapeDtypeStruct((NPAD, 128), f32)] * 2,
    )(xw10, acc1, deg_parts, b1p, W20p, W21p)

    acc2 = _scatter_128(u2, pk3)

    out = pl.pallas_call(
        _d2_body,
        grid=(_GRID,),
        in_specs=[_rows(128), _acc_spec(128), _DP_SPEC, _full((1, 128)),
                  _full((128, 512)), _full((1, 512))],
        out_specs=_rows(512),
        out_shape=jax.ShapeDtypeStruct((NPAD, 512), f32),
    )(y1w20, acc2, deg_parts, b2p, Wl, blp)

    return out[:N_NODES]
